# baseline (speedup 1.0000x reference)
import jax
import jax.numpy as jnp
from jax.experimental import pallas as pl
from jax.experimental.pallas import tpu as pltpu

_B = 16
_NC = 6


def _blank_body(cam_ref, imgs_ref, masks_ref, imgs_out_ref, masks_out_ref,
                zeros_ref, sem_img, sem_msk):
    zeros_ref[...] = jnp.zeros(zeros_ref.shape, jnp.float32)
    img_copies = []
    msk_copies = []
    for i in range(_B * _NC):
        b, c = divmod(i, _NC)
        keep = cam_ref[b] != c
        cp = pltpu.make_async_copy(imgs_ref.at[i], imgs_out_ref.at[i], sem_img)
        zp = pltpu.make_async_copy(zeros_ref, imgs_out_ref.at[i], sem_img)
        pl.when(keep)(cp.start)
        pl.when(jnp.logical_not(keep))(zp.start)
        img_copies.append(cp)
        cpm = pltpu.make_async_copy(masks_ref.at[i], masks_out_ref.at[i], sem_msk)
        zpm = pltpu.make_async_copy(zeros_ref.at[pl.ds(0, 128)], masks_out_ref.at[i], sem_msk)
        pl.when(keep)(cpm.start)
        pl.when(jnp.logical_not(keep))(zpm.start)
        msk_copies.append(cpm)
    for cp in img_copies:
        cp.wait()
    for cpm in msk_copies:
        cpm.wait()


def kernel(imgs, grids, masks, cam_choice):
    B, NC, C, H, W = imgs.shape
    imgs2 = imgs.reshape(B * NC, 384, 1152)
    masks2 = masks.reshape(B * NC, 128, 1152)

    imgs_out, masks_out = pl.pallas_call(
        _blank_body,
        grid_spec=pltpu.PrefetchScalarGridSpec(
            num_scalar_prefetch=1,
            grid=(1,),
            in_specs=[
                pl.BlockSpec(memory_space=pl.ANY),
                pl.BlockSpec(memory_space=pl.ANY),
            ],
            out_specs=[
                pl.BlockSpec(memory_space=pl.ANY),
                pl.BlockSpec(memory_space=pl.ANY),
            ],
            scratch_shapes=[
                pltpu.VMEM((384, 1152), jnp.float32),
                pltpu.SemaphoreType.DMA,
                pltpu.SemaphoreType.DMA,
            ],
        ),
        out_shape=[
            jax.ShapeDtypeStruct(imgs2.shape, imgs2.dtype),
            jax.ShapeDtypeStruct(masks2.shape, masks2.dtype),
        ],
    )(cam_choice.astype(jnp.int32), imgs2, masks2)

    return (imgs_out.reshape(imgs.shape), grids, masks_out.reshape(masks.shape))


# speedup vs baseline: 8.5348x; 8.5348x over previous
import functools

import jax
import jax.numpy as jnp
from jax import lax
from jax.experimental import pallas as pl
from jax.experimental.pallas import tpu as pltpu, tpu_sc as plsc

_B = 16
_NC = 6
_NW = 32
_CH = 36864
_IMG_ROW = 442368
_MSK_ROW = 147456
_SLICES_PER_W = (_B * _NC) // _NW


def _stream_row(row_in, row_out, nch, bufs, zeros, sem_in, sem_out, keep):
    nbuf = len(bufs)

    def _kept():
        ins = []
        outs = []
        for j in range(min(nbuf, nch)):
            cp = pltpu.make_async_copy(
                row_in.at[pl.ds(j * _CH, _CH)], bufs[j % nbuf], sem_in)
            cp.start()
            ins.append(cp)
        pending_out = []
        for j in range(nch):
            ins[j].wait()
            out = pltpu.make_async_copy(
                bufs[j % nbuf], row_out.at[pl.ds(j * _CH, _CH)], sem_out)
            out.start()
            pending_out.append(out)
            nxt = j + nbuf
            if nxt < nch:
                pending_out.pop(0).wait()
                cp = pltpu.make_async_copy(
                    row_in.at[pl.ds(nxt * _CH, _CH)], bufs[nxt % nbuf], sem_in)
                cp.start()
                ins.append(cp)
        for out in pending_out:
            out.wait()

    def _blank():
        outs = []
        for j in range(nch):
            out = pltpu.make_async_copy(
                zeros, row_out.at[pl.ds(j * _CH, _CH)], sem_out)
            out.start()
            outs.append(out)
        for out in outs:
            out.wait()

    pl.when(keep != 0)(_kept)
    pl.when(keep == 0)(_blank)


def _sc_body(cam_hbm, imgs_hbm, masks_hbm, imgs_out, masks_out,
             cam_v, buf0, buf1, zeros, sem_cam, sem_in, sem_out):
    wid = lax.axis_index("s") * 2 + lax.axis_index("c")

    pltpu.make_async_copy(cam_hbm, cam_v.at[pl.ds(0, 16)], sem_cam).start()

    def _zf(i, _):
        zeros[pl.ds(i * 16, 16)] = jnp.zeros((16,), jnp.float32)
        return 0
    lax.fori_loop(0, _CH // 16, _zf, 0)

    pltpu.make_async_copy(cam_hbm, cam_v.at[pl.ds(0, 16)], sem_cam).wait()
    bufs = (buf0, buf1)

    for k in range(_SLICES_PER_W):
        s = wid * _SLICES_PER_W + k
        b = s // _NC
        c = s % _NC
        cam_b = cam_v[pl.ds(b, 16)][0]
        keep = jnp.where(cam_b != c, 1, 0)
        _stream_row(imgs_hbm.at[s], imgs_out.at[s], _IMG_ROW // _CH,
                    bufs, zeros, sem_in, sem_out, keep)
        _stream_row(masks_hbm.at[s], masks_out.at[s], _MSK_ROW // _CH,
                    bufs, zeros, sem_in, sem_out, keep)


def kernel(imgs, grids, masks, cam_choice):
    B, NC, C, H, W = imgs.shape
    imgs2 = imgs.reshape(B * NC, C * H * W)
    masks2 = masks.reshape(B * NC, H * W)

    mesh = plsc.VectorSubcoreMesh(core_axis_name="c", subcore_axis_name="s")
    sc = functools.partial(
        pl.kernel,
        out_type=[
            jax.ShapeDtypeStruct(imgs2.shape, imgs2.dtype),
            jax.ShapeDtypeStruct(masks2.shape, masks2.dtype),
        ],
        mesh=mesh,
        scratch_types=[
            pltpu.VMEM((32,), jnp.int32),
            pltpu.VMEM((_CH,), jnp.float32),
            pltpu.VMEM((_CH,), jnp.float32),
            pltpu.VMEM((_CH,), jnp.float32),
            pltpu.SemaphoreType.DMA,
            pltpu.SemaphoreType.DMA,
            pltpu.SemaphoreType.DMA,
        ],
    )(_sc_body)

    imgs_out, masks_out = sc(cam_choice.astype(jnp.int32), imgs2, masks2)
    return (imgs_out.reshape(imgs.shape), grids, masks_out.reshape(masks.shape))


# speedup vs baseline: 8.5937x; 1.0069x over previous
import functools

import jax
import jax.numpy as jnp
from jax import lax
from jax.experimental import pallas as pl
from jax.experimental.pallas import tpu as pltpu, tpu_sc as plsc

_B = 16
_NC = 6
_NW = 32
_CH = 36864
_ZCH = 16384
_IMG_ROW = 442368
_MSK_ROW = 147456
_SLICES_PER_W = (_B * _NC) // _NW


def _stream_row(row_in, row_out, nch, bufs, zeros, sem_in, sem_out, keep):
    nbuf = len(bufs)

    def _kept():
        ins = {}
        outs = {}

        def start_in(j):
            d = pltpu.make_async_copy(
                row_in.at[pl.ds(j * _CH, _CH)], bufs[j % nbuf], sem_in)
            d.start()
            ins[j] = d

        def start_out(j):
            d = pltpu.make_async_copy(
                bufs[j % nbuf], row_out.at[pl.ds(j * _CH, _CH)], sem_out)
            d.start()
            outs[j] = d

        waited = set()
        for j in range(min(nbuf, nch)):
            start_in(j)
        for j in range(nch):
            if j >= 1 and (j + nbuf - 1) < nch:
                outs[j - 1].wait()
                waited.add(j - 1)
                start_in(j + nbuf - 1)
            ins[j].wait()
            start_out(j)
        for j in range(nch):
            if j not in waited:
                outs[j].wait()

    def _blank():
        outz = []
        nz = (nch * _CH) // _ZCH
        for j in range(nz):
            out = pltpu.make_async_copy(
                zeros, row_out.at[pl.ds(j * _ZCH, _ZCH)], sem_out)
            out.start()
            outz.append(out)
        for out in outz:
            out.wait()

    pl.when(keep != 0)(_kept)
    pl.when(keep == 0)(_blank)


def _sc_body(cam_hbm, imgs_hbm, masks_hbm, imgs_out, masks_out,
             cam_v, buf0, buf1, buf2, zeros, sem_cam, sem_in, sem_out):
    wid = lax.axis_index("s") * 2 + lax.axis_index("c")

    pltpu.make_async_copy(cam_hbm, cam_v.at[pl.ds(0, 16)], sem_cam).start()

    def _zf(i, _):
        zeros[pl.ds(i * 16, 16)] = jnp.zeros((16,), jnp.float32)
        return 0
    lax.fori_loop(0, _ZCH // 16, _zf, 0)

    pltpu.make_async_copy(cam_hbm, cam_v.at[pl.ds(0, 16)], sem_cam).wait()
    bufs = (buf0, buf1, buf2)

    for k in range(_SLICES_PER_W):
        s = wid * _SLICES_PER_W + k
        b = s // _NC
        c = s % _NC
        cam_b = cam_v[pl.ds(b, 16)][0]
        keep = jnp.where(cam_b != c, 1, 0)
        _stream_row(imgs_hbm.at[s], imgs_out.at[s], _IMG_ROW // _CH,
                    bufs, zeros, sem_in, sem_out, keep)
        _stream_row(masks_hbm.at[s], masks_out.at[s], _MSK_ROW // _CH,
                    bufs, zeros, sem_in, sem_out, keep)


def kernel(imgs, grids, masks, cam_choice):
    B, NC, C, H, W = imgs.shape
    imgs2 = imgs.reshape(B * NC, C * H * W)
    masks2 = masks.reshape(B * NC, H * W)

    mesh = plsc.VectorSubcoreMesh(core_axis_name="c", subcore_axis_name="s")
    sc = functools.partial(
        pl.kernel,
        out_type=[
            jax.ShapeDtypeStruct(imgs2.shape, imgs2.dtype),
            jax.ShapeDtypeStruct(masks2.shape, masks2.dtype),
        ],
        mesh=mesh,
        scratch_types=[
            pltpu.VMEM((32,), jnp.int32),
            pltpu.VMEM((_CH,), jnp.float32),
            pltpu.VMEM((_CH,), jnp.float32),
            pltpu.VMEM((_CH,), jnp.float32),
            pltpu.VMEM((_ZCH,), jnp.float32),
            pltpu.SemaphoreType.DMA,
            pltpu.SemaphoreType.DMA,
            pltpu.SemaphoreType.DMA,
        ],
    )(_sc_body)

    imgs_out, masks_out = sc(cam_choice.astype(jnp.int32), imgs2, masks2)
    return (imgs_out.reshape(imgs.shape), grids, masks_out.reshape(masks.shape))


# speedup vs baseline: 9.4547x; 1.1002x over previous
import functools

import jax
import jax.numpy as jnp
from jax.experimental import pallas as pl
from jax.experimental.pallas import tpu as pltpu

_B = 16
_NC = 6
_ROWS = _B * _NC
_CW = 147456
_IMG_CHUNKS = _ROWS * 3
_MSK_CHUNKS = _ROWS
_TOT = _IMG_CHUNKS + _MSK_CHUNKS
_NCHAN = 8
_NBUF = 3


def _tc_body(cam_ref, imgs_ref, masks_ref, imgs_out, masks_out, *scr):
    bufs = scr[:_NCHAN * _NBUF]
    zeros = scr[_NCHAN * _NBUF]
    sem_in = scr[_NCHAN * _NBUF + 1:_NCHAN * _NBUF + 1 + _NCHAN]
    sem_out = scr[_NCHAN * _NBUF + 1 + _NCHAN:]

    zeros[...] = jnp.zeros(zeros.shape, jnp.float32)

    def chunk_refs(g):
        if g < _IMG_CHUNKS:
            row = g // 3
            src = imgs_ref.at[g]
            dst = imgs_out.at[g]
        else:
            row = g - _IMG_CHUNKS
            src = masks_ref.at[row]
            dst = masks_out.at[row]
        keep = cam_ref[row // _NC] != (row % _NC)
        return src, dst, keep

    chans = [[g for g in range(_TOT) if g % _NCHAN == ch] for ch in range(_NCHAN)]
    n_per = len(chans[0])

    ins = [dict() for _ in range(_NCHAN)]
    outs = [dict() for _ in range(_NCHAN)]

    def start_in(ch, j):
        src, dst, keep = chunk_refs(chans[ch][j])
        buf = bufs[ch * _NBUF + j % _NBUF]
        d = pltpu.make_async_copy(src, buf, sem_in[ch])
        dz = pltpu.make_async_copy(zeros, buf, sem_in[ch])
        pl.when(keep)(d.start)
        pl.when(jnp.logical_not(keep))(dz.start)
        ins[ch][j] = d

    def start_out(ch, j):
        src, dst, keep = chunk_refs(chans[ch][j])
        buf = bufs[ch * _NBUF + j % _NBUF]
        d = pltpu.make_async_copy(buf, dst, sem_out[ch])
        d.start()
        outs[ch][j] = d

    for j in range(_NBUF):
        for ch in range(_NCHAN):
            start_in(ch, j)
    waited = [set() for _ in range(_NCHAN)]
    for j in range(n_per):
        for ch in range(_NCHAN):
            if j >= 1 and (j + _NBUF - 1) < n_per:
                outs[ch][j - 1].wait()
                waited[ch].add(j - 1)
                start_in(ch, j + _NBUF - 1)
            ins[ch][j].wait()
            start_out(ch, j)
    for ch in range(_NCHAN):
        for j in range(n_per):
            if j not in waited[ch]:
                outs[ch][j].wait()


def kernel(imgs, grids, masks, cam_choice):
    B, NC, C, H, W = imgs.shape
    imgs2 = imgs.reshape(B * NC * 3, 128, 1152)
    masks2 = masks.reshape(B * NC, 128, 1152)

    scratch = [pltpu.VMEM((128, 1152), jnp.float32)] * (_NCHAN * _NBUF)
    scratch += [pltpu.VMEM((128, 1152), jnp.float32)]
    scratch += [pltpu.SemaphoreType.DMA] * (2 * _NCHAN)

    imgs_out, masks_out = pl.pallas_call(
        _tc_body,
        grid_spec=pltpu.PrefetchScalarGridSpec(
            num_scalar_prefetch=1,
            grid=(1,),
            in_specs=[
                pl.BlockSpec(memory_space=pl.ANY),
                pl.BlockSpec(memory_space=pl.ANY),
            ],
            out_specs=[
                pl.BlockSpec(memory_space=pl.ANY),
                pl.BlockSpec(memory_space=pl.ANY),
            ],
            scratch_shapes=scratch,
        ),
        out_shape=[
            jax.ShapeDtypeStruct(imgs2.shape, imgs2.dtype),
            jax.ShapeDtypeStruct(masks2.shape, masks2.dtype),
        ],
    )(cam_choice.astype(jnp.int32), imgs2, masks2)

    return (imgs_out.reshape(imgs.shape), grids, masks_out.reshape(masks.shape))


# speedup vs baseline: 10.0604x; 1.0641x over previous
import functools

import jax
import jax.numpy as jnp
from jax import lax
from jax.experimental import pallas as pl
from jax.experimental.pallas import tpu as pltpu, tpu_sc as plsc

_B = 16
_NC = 6
_CW = 73728
_NISS = 8
_IMG_CHUNKS_PER_CORE = 288
_CHUNKS_PER_CORE = 384
_NGROUPS = _CHUNKS_PER_CORE // _NISS
_ZSTAGE = 73728


def _sc_body(cam_hbm, imgs_hbm, masks_hbm, imgs_out, masks_out,
             cam_v, zstage, slots, zeros, sem_cam, sem_z, sem_in, sem_out):
    cid = lax.axis_index("c")
    sid = lax.axis_index("s")

    pltpu.make_async_copy(cam_hbm, cam_v.at[pl.ds(0, 16)], sem_cam).start()

    @pl.when(sid == 0)
    def _init_zeros():
        def _zf(i, _):
            zstage[pl.ds(i * 16, 16)] = jnp.zeros((16,), jnp.float32)
            return 0
        lax.fori_loop(0, _ZSTAGE // 16, _zf, 0)
        pltpu.make_async_copy(zstage, zeros, sem_z).start()
        pltpu.make_async_copy(zstage, zeros, sem_z).wait()

    pltpu.make_async_copy(cam_hbm, cam_v.at[pl.ds(0, 16)], sem_cam).wait()
    plsc.subcore_barrier()

    @pl.when(sid < _NISS)
    def _issue():
        slot = slots.at[sid]

        def _group(g, _):
            ch = g * _NISS + sid

            def _do(row, src_row, dst_row):
                b = row // _NC
                c = row % _NC
                keep = cam_v[pl.ds(b, 16)][0] != c

                @pl.when(keep)
                def _copy():
                    pltpu.make_async_copy(src_row, slot, sem_in).start()
                    pltpu.make_async_copy(src_row, slot, sem_in).wait()
                    pltpu.make_async_copy(slot, dst_row, sem_out).start()
                    pltpu.make_async_copy(slot, dst_row, sem_out).wait()

                @pl.when(jnp.logical_not(keep))
                def _blank():
                    pltpu.make_async_copy(zeros, dst_row, sem_out).start()
                    pltpu.make_async_copy(zeros, dst_row, sem_out).wait()

            @pl.when(ch < _IMG_CHUNKS_PER_CORE)
            def _img():
                idx = cid * _IMG_CHUNKS_PER_CORE + ch
                _do(idx // 6, imgs_hbm.at[idx], imgs_out.at[idx])

            @pl.when(ch >= _IMG_CHUNKS_PER_CORE)
            def _msk():
                idx = cid * 96 + (ch - _IMG_CHUNKS_PER_CORE)
                _do(idx // 2, masks_hbm.at[idx], masks_out.at[idx])

            return 0

        lax.fori_loop(0, _NGROUPS, _group, 0)


def kernel(imgs, grids, masks, cam_choice):
    B, NC, C, H, W = imgs.shape
    imgs3 = imgs.reshape(B * NC * 6, _CW)
    masks2 = masks.reshape(B * NC * 2, _CW)

    mesh = plsc.VectorSubcoreMesh(core_axis_name="c", subcore_axis_name="s")
    sc = functools.partial(
        pl.kernel,
        out_type=[
            jax.ShapeDtypeStruct(imgs3.shape, imgs3.dtype),
            jax.ShapeDtypeStruct(masks2.shape, masks2.dtype),
        ],
        mesh=mesh,
        scratch_types=[
            pltpu.VMEM((32,), jnp.int32),
            pltpu.VMEM((_ZSTAGE,), jnp.float32),
            pltpu.MemorySpace.VMEM_SHARED((_NISS, _CW), jnp.float32),
            pltpu.MemorySpace.VMEM_SHARED((_CW,), jnp.float32),
            pltpu.SemaphoreType.DMA,
            pltpu.SemaphoreType.DMA,
            pltpu.SemaphoreType.DMA,
            pltpu.SemaphoreType.DMA,
        ],
    )(_sc_body)

    imgs_out, masks_out = sc(cam_choice.astype(jnp.int32), imgs3, masks2)
    return (imgs_out.reshape(imgs.shape), grids, masks_out.reshape(masks.shape))
